# baseline (speedup 1.0000x reference)
import functools
import jax
import jax.numpy as jnp
from jax.experimental import pallas as pl
from jax.experimental.pallas import tpu as pltpu

N = 2048
D = 768
H = 3072
O = 768
E = 8
K = 2

BN = 256
NEG = -1e30


def _gate_body(x_ref, g1_ref, gb1_ref, g2_ref, gb2_ref, g3_ref, gb3_ref,
               p_ref):
    x = x_ref[...].astype(jnp.bfloat16)
    h1 = jnp.maximum(
        jnp.dot(x, g1_ref[...], preferred_element_type=jnp.float32)
        + gb1_ref[...], 0.0).astype(jnp.bfloat16)
    h2 = jnp.maximum(
        jnp.dot(h1, g2_ref[...], preferred_element_type=jnp.float32)
        + gb2_ref[...], 0.0).astype(jnp.bfloat16)
    logits = jnp.dot(h2, g3_ref[...], preferred_element_type=jnp.float32) \
        + gb3_ref[...]
    lane = jax.lax.broadcasted_iota(jnp.int32, logits.shape, 1)
    lm = jnp.where(lane < E, logits, NEG)
    m1 = jnp.max(lm, axis=1, keepdims=True)
    i1 = jnp.min(jnp.where(lm == m1, lane, 10**6), axis=1, keepdims=True)
    lm2 = jnp.where(lane == i1, NEG, lm)
    m2 = jnp.max(lm2, axis=1, keepdims=True)
    i2 = jnp.min(jnp.where(lm2 == m2, lane, 10**6), axis=1, keepdims=True)
    ed = jnp.exp(m2 - m1)
    denom = 1.0 + ed
    p1 = 1.0 / denom
    p2 = ed / denom
    p = jnp.where(lane == i1, p1, jnp.where(lane == i2, p2, 0.0))
    p_ref[...] = p[:, :E]


def _expert_body(p_ref, x_ref, w1_ref, b1_ref, w2_ref, b2_ref, y_ref,
                 acc_ref):
    e = pl.program_id(0)
    i = pl.program_id(1)
    xb = x_ref[...]
    he = jnp.dot(xb, w1_ref[0], preferred_element_type=jnp.float32)
    he = jnp.maximum(he + b1_ref[0], 0.0).astype(jnp.bfloat16)
    out = jnp.dot(he, w2_ref[0], preferred_element_type=jnp.float32)
    out = out + b2_ref[0]
    oh = (jax.lax.broadcasted_iota(jnp.int32, (E, O), 0) == e).astype(jnp.float32)
    pe = jnp.dot(p_ref[...], oh, preferred_element_type=jnp.float32)
    rows = pl.ds(i * BN, BN)

    @pl.when(e == 0)
    def _():
        acc_ref[rows, :] = pe * out

    @pl.when(e > 0)
    def _():
        acc_ref[rows, :] = acc_ref[rows, :] + pe * out

    y_ref[...] = acc_ref[rows, :]


@jax.jit
def kernel(x, W1, b1, W2, b2, g1, gb1, g2, gb2, g3, gb3):
    g3p = jnp.zeros((D, 128), jnp.float32).at[:, :E].set(g3).astype(jnp.bfloat16)
    gb3p = jnp.zeros((1, 128), jnp.float32).at[0, :E].set(gb3)
    g1bf = g1.astype(jnp.bfloat16)
    g2bf = g2.astype(jnp.bfloat16)

    p = pl.pallas_call(
        _gate_body,
        grid=(N // BN,),
        in_specs=[
            pl.BlockSpec((BN, D), lambda i: (i, 0)),
            pl.BlockSpec((D, 4 * D), lambda i: (0, 0)),
            pl.BlockSpec((1, 4 * D), lambda i: (0, 0)),
            pl.BlockSpec((4 * D, D), lambda i: (0, 0)),
            pl.BlockSpec((1, D), lambda i: (0, 0)),
            pl.BlockSpec((D, 128), lambda i: (0, 0)),
            pl.BlockSpec((1, 128), lambda i: (0, 0)),
        ],
        out_specs=pl.BlockSpec((BN, E), lambda i: (i, 0)),
        out_shape=jax.ShapeDtypeStruct((N, E), jnp.float32),
    )(x, g1bf, gb1.reshape(1, 4 * D), g2bf, gb2.reshape(1, D), g3p, gb3p)

    xbf = x.astype(jnp.bfloat16)
    w1bf = W1.astype(jnp.bfloat16)
    w2bf = W2.astype(jnp.bfloat16)
    b1r = b1.reshape(E, 1, H)
    b2r = b2.reshape(E, 1, O)

    y = pl.pallas_call(
        _expert_body,
        grid=(E, N // BN),
        in_specs=[
            pl.BlockSpec((BN, E), lambda e, i: (i, 0)),
            pl.BlockSpec((BN, D), lambda e, i: (i, 0)),
            pl.BlockSpec((1, D, H), lambda e, i: (e, 0, 0)),
            pl.BlockSpec((1, 1, H), lambda e, i: (e, 0, 0)),
            pl.BlockSpec((1, H, O), lambda e, i: (e, 0, 0)),
            pl.BlockSpec((1, 1, O), lambda e, i: (e, 0, 0)),
        ],
        out_specs=pl.BlockSpec((BN, O), lambda e, i: (i, 0)),
        out_shape=jax.ShapeDtypeStruct((N, O), jnp.float32),
        scratch_shapes=[pltpu.VMEM((N, O), jnp.float32)],
    )(p, xbf, w1bf, b1r, w2bf, b2r)

    return (y, p)


# speedup vs baseline: 1.2223x; 1.2223x over previous
import functools
import jax
import jax.numpy as jnp
from jax import lax
from jax.experimental import pallas as pl
from jax.experimental.pallas import tpu as pltpu
from jax.experimental.pallas import tpu_sc as plsc

N = 2048
D = 768
H = 3072
O = 768
E = 8
K = 2

BN = 256
BM = 128
SPAD = N * K + E * BM
NB = SPAD // BM
SB = 512
NEG = -1e30

NC, NS = 2, 16
NW = NC * NS


def _gate_body(x_ref, g1_ref, gb1_ref, g2_ref, gb2_ref, g3_ref, gb3_ref,
               p_ref, ti_ref, tp_ref):
    x = x_ref[...].astype(jnp.bfloat16)
    h1 = jnp.maximum(
        jnp.dot(x, g1_ref[...], preferred_element_type=jnp.float32)
        + gb1_ref[...], 0.0).astype(jnp.bfloat16)
    h2 = jnp.maximum(
        jnp.dot(h1, g2_ref[...], preferred_element_type=jnp.float32)
        + gb2_ref[...], 0.0).astype(jnp.bfloat16)
    logits = jnp.dot(h2, g3_ref[...], preferred_element_type=jnp.float32) \
        + gb3_ref[...]
    lane = jax.lax.broadcasted_iota(jnp.int32, logits.shape, 1)
    lm = jnp.where(lane < E, logits, NEG)
    m1 = jnp.max(lm, axis=1, keepdims=True)
    i1 = jnp.min(jnp.where(lm == m1, lane, 10**6), axis=1, keepdims=True)
    lm2 = jnp.where(lane == i1, NEG, lm)
    m2 = jnp.max(lm2, axis=1, keepdims=True)
    i2 = jnp.min(jnp.where(lm2 == m2, lane, 10**6), axis=1, keepdims=True)
    ed = jnp.exp(m2 - m1)
    denom = 1.0 + ed
    p1 = 1.0 / denom
    p2 = ed / denom
    p = jnp.where(lane == i1, p1, jnp.where(lane == i2, p2, 0.0))
    p_ref[...] = p[:, :E]
    ti_ref[:, 0:1] = i1
    ti_ref[:, 1:2] = i2
    tp_ref[:, 0:1] = p1
    tp_ref[:, 1:2] = p2


def _rank_body(ti0_ref, ti1_ref, pos_ref, blk_ref):
    e_row = jax.lax.broadcasted_iota(jnp.int32, (1, E), 1)
    ir = jax.lax.broadcasted_iota(jnp.int32, (SB, SB), 0)
    ic = jax.lax.broadcasted_iota(jnp.int32, (SB, SB), 1)
    ltri = (ir > ic).astype(jnp.bfloat16)
    m8r = jax.lax.broadcasted_iota(jnp.int32, (E, E), 0)
    m8c = jax.lax.broadcasted_iota(jnp.int32, (E, E), 1)
    mincl = (m8r <= m8c).astype(jnp.float32)

    def ind(ref, j):
        col = ref[pl.ds(j * SB, SB), :]
        return (col == e_row).astype(jnp.float32)

    def cnt0(j, c):
        return c + jnp.sum(ind(ti0_ref, j), axis=0, keepdims=True)

    def cnt1(j, c):
        return c + jnp.sum(ind(ti1_ref, j), axis=0, keepdims=True)

    c = jnp.zeros((1, E), jnp.float32)
    c = lax.fori_loop(0, N // SB, cnt0, c)
    c = lax.fori_loop(0, N // SB, cnt1, c)
    gp = ((c.astype(jnp.int32) + BM - 1) // BM) * BM
    gpf = gp.astype(jnp.float32)
    cumgp = jnp.dot(gpf, mincl, preferred_element_type=jnp.float32)
    gs = cumgp - gpf

    jbase = jax.lax.broadcasted_iota(jnp.int32, (1, 64), 1) * BM
    cum_i = cumgp.astype(jnp.int32)
    acc = jnp.zeros((1, 64), jnp.int32)
    for e in range(E):
        acc = acc + (jbase >= cum_i[0:1, e:e + 1]).astype(jnp.int32)
    blk_ref[...] = jnp.minimum(acc, E - 1)

    def rank(ref, kk, j, c):
        ib = ind(ref, j)
        r = jnp.dot(ltri, ib.astype(jnp.bfloat16),
                    preferred_element_type=jnp.float32) + c
        slot = jnp.sum(ib * (gs + r), axis=1, keepdims=True)
        pos_ref[pl.ds(j * SB, SB), kk:kk + 1] = slot.astype(jnp.int32)
        return c + jnp.sum(ib, axis=0, keepdims=True)

    c2 = jnp.zeros((1, E), jnp.float32)
    c2 = lax.fori_loop(0, N // SB, functools.partial(rank, ti0_ref, 0), c2)
    c2 = lax.fori_loop(0, N // SB, functools.partial(rank, ti1_ref, 1), c2)


_CH3 = (N * K) // NW


@functools.lru_cache(maxsize=None)
def _make_sc_scatter():
    @functools.partial(
        pl.kernel,
        mesh=plsc.VectorSubcoreMesh(core_axis_name="c", subcore_axis_name="s"),
        out_type=jax.ShapeDtypeStruct((SPAD, D), jnp.float32),
        scratch_types=[
            pltpu.VMEM((_CH3,), jnp.int32),
            pltpu.VMEM((_CH3, D), jnp.float32),
            pltpu.SemaphoreType.DMA,
        ],
    )
    def body(x_hbm, pos_hbm, xs_hbm, idx_v, rows_v, sem):
        wid = lax.axis_index("s") * NC + lax.axis_index("c")
        base = wid * _CH3
        kq = base // N
        t0 = base - kq * N
        pltpu.sync_copy(pos_hbm.at[kq, pl.ds(t0, _CH3)], idx_v)
        pltpu.sync_copy(x_hbm.at[pl.ds(t0, _CH3), :], rows_v)
        pltpu.async_copy(rows_v, xs_hbm.at[idx_v], sem).wait()

    return body


def _sc_scatter_rows(x, posT):
    return _make_sc_scatter()(x, posT)


def _expert_body(s_ref, xs_ref, w1_ref, b1_ref, w2_ref, b2_ref, out_ref):
    xb = xs_ref[...].astype(jnp.bfloat16)
    he = jnp.dot(xb, w1_ref[0], preferred_element_type=jnp.float32)
    he = jnp.maximum(he + b1_ref[0], 0.0).astype(jnp.bfloat16)
    out = jnp.dot(he, w2_ref[0], preferred_element_type=jnp.float32)
    out_ref[...] = out + b2_ref[0]


_CH5 = N // NW


@functools.lru_cache(maxsize=None)
def _make_sc_gather():
    @functools.partial(
        pl.kernel,
        mesh=plsc.VectorSubcoreMesh(core_axis_name="c", subcore_axis_name="s"),
        out_type=(jax.ShapeDtypeStruct((N, O), jnp.float32),
                  jax.ShapeDtypeStruct((N, O), jnp.float32)),
        scratch_types=[
            pltpu.VMEM((_CH5,), jnp.int32),
            pltpu.VMEM((_CH5, O), jnp.float32),
            pltpu.SemaphoreType.DMA,
        ],
    )
    def body(rows_hbm, pos_hbm, r0_hbm, r1_hbm, idx_v, buf_v, sem):
        wid = lax.axis_index("s") * NC + lax.axis_index("c")
        t0 = wid * _CH5
        pltpu.sync_copy(pos_hbm.at[0, pl.ds(t0, _CH5)], idx_v)
        pltpu.async_copy(rows_hbm.at[idx_v], buf_v, sem).wait()
        pltpu.sync_copy(buf_v, r0_hbm.at[pl.ds(t0, _CH5), :])
        pltpu.sync_copy(pos_hbm.at[1, pl.ds(t0, _CH5)], idx_v)
        pltpu.async_copy(rows_hbm.at[idx_v], buf_v, sem).wait()
        pltpu.sync_copy(buf_v, r1_hbm.at[pl.ds(t0, _CH5), :])

    return body


def _sc_gather_out(rows, posT):
    return _make_sc_gather()(rows, posT)


def _combine_body(tp_ref, r0_ref, r1_ref, y_ref):
    y_ref[...] = tp_ref[:, 0:1] * r0_ref[...] + tp_ref[:, 1:2] * r1_ref[...]


@jax.jit
def kernel(x, W1, b1, W2, b2, g1, gb1, g2, gb2, g3, gb3):
    g3p = jnp.zeros((D, 128), jnp.float32).at[:, :E].set(g3).astype(jnp.bfloat16)
    gb3p = jnp.zeros((1, 128), jnp.float32).at[0, :E].set(gb3)
    g1bf = g1.astype(jnp.bfloat16)
    g2bf = g2.astype(jnp.bfloat16)

    p, topi, topp = pl.pallas_call(
        _gate_body,
        grid=(N // BN,),
        in_specs=[
            pl.BlockSpec((BN, D), lambda i: (i, 0)),
            pl.BlockSpec((D, 4 * D), lambda i: (0, 0)),
            pl.BlockSpec((1, 4 * D), lambda i: (0, 0)),
            pl.BlockSpec((4 * D, D), lambda i: (0, 0)),
            pl.BlockSpec((1, D), lambda i: (0, 0)),
            pl.BlockSpec((D, 128), lambda i: (0, 0)),
            pl.BlockSpec((1, 128), lambda i: (0, 0)),
        ],
        out_specs=[
            pl.BlockSpec((BN, E), lambda i: (i, 0)),
            pl.BlockSpec((BN, K), lambda i: (i, 0)),
            pl.BlockSpec((BN, K), lambda i: (i, 0)),
        ],
        out_shape=[
            jax.ShapeDtypeStruct((N, E), jnp.float32),
            jax.ShapeDtypeStruct((N, K), jnp.int32),
            jax.ShapeDtypeStruct((N, K), jnp.float32),
        ],
    )(x, g1bf, gb1.reshape(1, 4 * D), g2bf, gb2.reshape(1, D), g3p, gb3p)

    ti0 = topi[:, 0:1]
    ti1 = topi[:, 1:2]
    pos, blk = pl.pallas_call(
        _rank_body,
        grid=(1,),
        in_specs=[
            pl.BlockSpec((N, 1), lambda i: (0, 0)),
            pl.BlockSpec((N, 1), lambda i: (0, 0)),
        ],
        out_specs=[
            pl.BlockSpec((N, K), lambda i: (0, 0)),
            pl.BlockSpec((1, 64), lambda i: (0, 0)),
        ],
        out_shape=[
            jax.ShapeDtypeStruct((N, K), jnp.int32),
            jax.ShapeDtypeStruct((1, 64), jnp.int32),
        ],
    )(ti0, ti1)

    posT = pos.T
    blk1d = blk.reshape(64)[:NB]

    xs = _sc_scatter_rows(x, posT)

    w1bf = W1.astype(jnp.bfloat16)
    w2bf = W2.astype(jnp.bfloat16)
    b1r = b1.reshape(E, 1, H)
    b2r = b2.reshape(E, 1, O)
    out_rows = pl.pallas_call(
        _expert_body,
        grid_spec=pltpu.PrefetchScalarGridSpec(
            num_scalar_prefetch=1,
            grid=(NB,),
            in_specs=[
                pl.BlockSpec((BM, D), lambda j, s: (j, 0)),
                pl.BlockSpec((1, D, H), lambda j, s: (s[j], 0, 0)),
                pl.BlockSpec((1, 1, H), lambda j, s: (s[j], 0, 0)),
                pl.BlockSpec((1, H, O), lambda j, s: (s[j], 0, 0)),
                pl.BlockSpec((1, 1, O), lambda j, s: (s[j], 0, 0)),
            ],
            out_specs=pl.BlockSpec((BM, O), lambda j, s: (j, 0)),
        ),
        out_shape=jax.ShapeDtypeStruct((SPAD, O), jnp.float32),
    )(blk1d, xs, w1bf, b1r, w2bf, b2r)

    r0, r1 = _sc_gather_out(out_rows, posT)

    y = pl.pallas_call(
        _combine_body,
        grid=(N // BN,),
        in_specs=[
            pl.BlockSpec((BN, K), lambda i: (i, 0)),
            pl.BlockSpec((BN, O), lambda i: (i, 0)),
            pl.BlockSpec((BN, O), lambda i: (i, 0)),
        ],
        out_specs=pl.BlockSpec((BN, O), lambda i: (i, 0)),
        out_shape=jax.ShapeDtypeStruct((N, O), jnp.float32),
    )(topp, r0, r1)

    return (y, p)


# speedup vs baseline: 1.5317x; 1.2532x over previous
import functools
import jax
import jax.numpy as jnp
from jax import lax
from jax.experimental import pallas as pl
from jax.experimental.pallas import tpu as pltpu
from jax.experimental.pallas import tpu_sc as plsc

N = 2048
D = 768
H = 3072
O = 768
E = 8
K = 2

BN = 256
BM = 128
SPAD = N * K + E * BM
NB = SPAD // BM
SB = 512
NEG = -1e30

NC, NS = 2, 16
NW = NC * NS


def _gate_body(x_ref, g1_ref, gb1_ref, g2_ref, gb2_ref, g3_ref, gb3_ref,
               p_ref, ti_ref, tp_ref):
    x = x_ref[...].astype(jnp.bfloat16)
    h1 = jnp.maximum(
        jnp.dot(x, g1_ref[...].astype(jnp.bfloat16),
                preferred_element_type=jnp.float32)
        + gb1_ref[...], 0.0).astype(jnp.bfloat16)
    h2 = jnp.maximum(
        jnp.dot(h1, g2_ref[...].astype(jnp.bfloat16),
                preferred_element_type=jnp.float32)
        + gb2_ref[...], 0.0).astype(jnp.bfloat16)
    logits = jnp.dot(h2, g3_ref[...].astype(jnp.bfloat16),
                     preferred_element_type=jnp.float32) \
        + gb3_ref[...]
    lane = jax.lax.broadcasted_iota(jnp.int32, logits.shape, 1)
    lm = jnp.where(lane < E, logits, NEG)
    m1 = jnp.max(lm, axis=1, keepdims=True)
    i1 = jnp.min(jnp.where(lm == m1, lane, 10**6), axis=1, keepdims=True)
    lm2 = jnp.where(lane == i1, NEG, lm)
    m2 = jnp.max(lm2, axis=1, keepdims=True)
    i2 = jnp.min(jnp.where(lm2 == m2, lane, 10**6), axis=1, keepdims=True)
    ed = jnp.exp(m2 - m1)
    denom = 1.0 + ed
    p1 = 1.0 / denom
    p2 = ed / denom
    p = jnp.where(lane == i1, p1, jnp.where(lane == i2, p2, 0.0))
    p_ref[...] = p[:, :E]
    ti_ref[:, 0:1] = i1
    ti_ref[:, 1:2] = i2
    tp_ref[:, 0:1] = p1
    tp_ref[:, 1:2] = p2


def _rank_body(ti0_ref, ti1_ref, pos_ref, blk_ref):
    e_row = jax.lax.broadcasted_iota(jnp.int32, (1, E), 1)
    ir = jax.lax.broadcasted_iota(jnp.int32, (SB, SB), 0)
    ic = jax.lax.broadcasted_iota(jnp.int32, (SB, SB), 1)
    ltri = (ir > ic).astype(jnp.bfloat16)
    m8r = jax.lax.broadcasted_iota(jnp.int32, (E, E), 0)
    m8c = jax.lax.broadcasted_iota(jnp.int32, (E, E), 1)
    mincl = (m8r <= m8c).astype(jnp.float32)

    def ind(ref, j):
        col = ref[pl.ds(j * SB, SB), :]
        return (col == e_row).astype(jnp.float32)

    def cnt0(j, c):
        return c + jnp.sum(ind(ti0_ref, j), axis=0, keepdims=True)

    def cnt1(j, c):
        return c + jnp.sum(ind(ti1_ref, j), axis=0, keepdims=True)

    c = jnp.zeros((1, E), jnp.float32)
    c = lax.fori_loop(0, N // SB, cnt0, c)
    c = lax.fori_loop(0, N // SB, cnt1, c)
    gp = ((c.astype(jnp.int32) + BM - 1) // BM) * BM
    gpf = gp.astype(jnp.float32)
    cumgp = jnp.dot(gpf, mincl, preferred_element_type=jnp.float32)
    gs = cumgp - gpf

    jbase = jax.lax.broadcasted_iota(jnp.int32, (1, 64), 1) * BM
    cum_i = cumgp.astype(jnp.int32)
    acc = jnp.zeros((1, 64), jnp.int32)
    for e in range(E):
        acc = acc + (jbase >= cum_i[0:1, e:e + 1]).astype(jnp.int32)
    blk_ref[...] = jnp.minimum(acc, E - 1)

    def rank(ref, kk, j, c):
        ib = ind(ref, j)
        r = jnp.dot(ltri, ib.astype(jnp.bfloat16),
                    preferred_element_type=jnp.float32) + c
        slot = jnp.sum(ib * (gs + r), axis=1, keepdims=True)
        pos_ref[pl.ds(j * SB, SB), kk:kk + 1] = slot.astype(jnp.int32)
        return c + jnp.sum(ib, axis=0, keepdims=True)

    c2 = jnp.zeros((1, E), jnp.float32)
    c2 = lax.fori_loop(0, N // SB, functools.partial(rank, ti0_ref, 0), c2)
    c2 = lax.fori_loop(0, N // SB, functools.partial(rank, ti1_ref, 1), c2)


_CH3 = (N * K) // NW


@functools.lru_cache(maxsize=None)
def _make_sc_scatter():
    @functools.partial(
        pl.kernel,
        mesh=plsc.VectorSubcoreMesh(core_axis_name="c", subcore_axis_name="s"),
        out_type=jax.ShapeDtypeStruct((SPAD, D), jnp.float32),
        scratch_types=[
            pltpu.VMEM((_CH3,), jnp.int32),
            pltpu.VMEM((_CH3, D), jnp.float32),
            pltpu.SemaphoreType.DMA,
        ],
    )
    def body(x_hbm, pos_hbm, xs_hbm, idx_v, rows_v, sem):
        wid = lax.axis_index("s") * NC + lax.axis_index("c")
        base = wid * _CH3
        kq = base // N
        t0 = base - kq * N
        pltpu.sync_copy(pos_hbm.at[kq, pl.ds(t0, _CH3)], idx_v)
        pltpu.sync_copy(x_hbm.at[pl.ds(t0, _CH3), :], rows_v)
        pltpu.async_copy(rows_v, xs_hbm.at[idx_v], sem).wait()

    return body


def _sc_scatter_rows(x, posT):
    return _make_sc_scatter()(x, posT)


def _expert_body(s_ref, xs_ref, w1_ref, b1_ref, w2_ref, b2_ref, out_ref):
    xb = xs_ref[...].astype(jnp.bfloat16)
    he = jnp.dot(xb, w1_ref[0].astype(jnp.bfloat16),
                 preferred_element_type=jnp.float32)
    he = jnp.maximum(he + b1_ref[0], 0.0).astype(jnp.bfloat16)
    out = jnp.dot(he, w2_ref[0].astype(jnp.bfloat16),
                  preferred_element_type=jnp.float32)
    out_ref[...] = out + b2_ref[0]


_CH5 = N // NW


@functools.lru_cache(maxsize=None)
def _make_sc_gather():
    @functools.partial(
        pl.kernel,
        mesh=plsc.VectorSubcoreMesh(core_axis_name="c", subcore_axis_name="s"),
        out_type=(jax.ShapeDtypeStruct((N, O), jnp.float32),
                  jax.ShapeDtypeStruct((N, O), jnp.float32)),
        scratch_types=[
            pltpu.VMEM((_CH5,), jnp.int32),
            pltpu.VMEM((_CH5, O), jnp.float32),
            pltpu.SemaphoreType.DMA,
        ],
    )
    def body(rows_hbm, pos_hbm, r0_hbm, r1_hbm, idx_v, buf_v, sem):
        wid = lax.axis_index("s") * NC + lax.axis_index("c")
        t0 = wid * _CH5
        pltpu.sync_copy(pos_hbm.at[0, pl.ds(t0, _CH5)], idx_v)
        pltpu.async_copy(rows_hbm.at[idx_v], buf_v, sem).wait()
        pltpu.sync_copy(buf_v, r0_hbm.at[pl.ds(t0, _CH5), :])
        pltpu.sync_copy(pos_hbm.at[1, pl.ds(t0, _CH5)], idx_v)
        pltpu.async_copy(rows_hbm.at[idx_v], buf_v, sem).wait()
        pltpu.sync_copy(buf_v, r1_hbm.at[pl.ds(t0, _CH5), :])

    return body


def _sc_gather_out(rows, posT):
    return _make_sc_gather()(rows, posT)


def _combine_body(tp_ref, r0_ref, r1_ref, y_ref):
    y_ref[...] = tp_ref[:, 0:1] * r0_ref[...] + tp_ref[:, 1:2] * r1_ref[...]


@jax.jit
def kernel(x, W1, b1, W2, b2, g1, gb1, g2, gb2, g3, gb3):
    g3p = jnp.zeros((D, 128), jnp.float32).at[:, :E].set(g3)
    gb3p = jnp.zeros((1, 128), jnp.float32).at[0, :E].set(gb3)

    p, topi, topp = pl.pallas_call(
        _gate_body,
        grid=(N // BN,),
        in_specs=[
            pl.BlockSpec((BN, D), lambda i: (i, 0)),
            pl.BlockSpec((D, 4 * D), lambda i: (0, 0)),
            pl.BlockSpec((1, 4 * D), lambda i: (0, 0)),
            pl.BlockSpec((4 * D, D), lambda i: (0, 0)),
            pl.BlockSpec((1, D), lambda i: (0, 0)),
            pl.BlockSpec((D, 128), lambda i: (0, 0)),
            pl.BlockSpec((1, 128), lambda i: (0, 0)),
        ],
        out_specs=[
            pl.BlockSpec((BN, E), lambda i: (i, 0)),
            pl.BlockSpec((BN, K), lambda i: (i, 0)),
            pl.BlockSpec((BN, K), lambda i: (i, 0)),
        ],
        out_shape=[
            jax.ShapeDtypeStruct((N, E), jnp.float32),
            jax.ShapeDtypeStruct((N, K), jnp.int32),
            jax.ShapeDtypeStruct((N, K), jnp.float32),
        ],
    )(x, g1, gb1.reshape(1, 4 * D), g2, gb2.reshape(1, D), g3p, gb3p)

    ti0 = topi[:, 0:1]
    ti1 = topi[:, 1:2]
    pos, blk = pl.pallas_call(
        _rank_body,
        grid=(1,),
        in_specs=[
            pl.BlockSpec((N, 1), lambda i: (0, 0)),
            pl.BlockSpec((N, 1), lambda i: (0, 0)),
        ],
        out_specs=[
            pl.BlockSpec((N, K), lambda i: (0, 0)),
            pl.BlockSpec((1, 64), lambda i: (0, 0)),
        ],
        out_shape=[
            jax.ShapeDtypeStruct((N, K), jnp.int32),
            jax.ShapeDtypeStruct((1, 64), jnp.int32),
        ],
    )(ti0, ti1)

    posT = pos.T
    blk1d = blk.reshape(64)[:NB]

    xs = _sc_scatter_rows(x, posT)

    b1r = b1.reshape(E, 1, H)
    b2r = b2.reshape(E, 1, O)
    out_rows = pl.pallas_call(
        _expert_body,
        grid_spec=pltpu.PrefetchScalarGridSpec(
            num_scalar_prefetch=1,
            grid=(NB,),
            in_specs=[
                pl.BlockSpec((BM, D), lambda j, s: (j, 0)),
                pl.BlockSpec((1, D, H), lambda j, s: (s[j], 0, 0)),
                pl.BlockSpec((1, 1, H), lambda j, s: (s[j], 0, 0)),
                pl.BlockSpec((1, H, O), lambda j, s: (s[j], 0, 0)),
                pl.BlockSpec((1, 1, O), lambda j, s: (s[j], 0, 0)),
            ],
            out_specs=pl.BlockSpec((BM, O), lambda j, s: (j, 0)),
        ),
        out_shape=jax.ShapeDtypeStruct((SPAD, O), jnp.float32),
    )(blk1d, xs, W1, b1r, W2, b2r)

    r0, r1 = _sc_gather_out(out_rows, posT)

    y = pl.pallas_call(
        _combine_body,
        grid=(N // BN,),
        in_specs=[
            pl.BlockSpec((BN, K), lambda i: (i, 0)),
            pl.BlockSpec((BN, O), lambda i: (i, 0)),
            pl.BlockSpec((BN, O), lambda i: (i, 0)),
        ],
        out_specs=pl.BlockSpec((BN, O), lambda i: (i, 0)),
        out_shape=jax.ShapeDtypeStruct((N, O), jnp.float32),
    )(topp, r0, r1)

    return (y, p)


# speedup vs baseline: 1.5700x; 1.0250x over previous
import functools
import jax
import jax.numpy as jnp
from jax import lax
from jax.experimental import pallas as pl
from jax.experimental.pallas import tpu as pltpu
from jax.experimental.pallas import tpu_sc as plsc

N = 2048
D = 768
H = 3072
O = 768
E = 8
K = 2

BN = 256
BM = 128
SPAD = N * K + E * BM
NB = SPAD // BM
SB = 512
NEG = -1e30

NC, NS = 2, 16
NW = NC * NS


def _gate_body(x_ref, g1_ref, gb1_ref, g2_ref, gb2_ref, g3_ref, gb3_ref,
               p_ref, tp_ref, pos_ref, blk_ref, ti_scr):
    x = x_ref[...].astype(jnp.bfloat16)
    h1 = jnp.maximum(
        jnp.dot(x, g1_ref[...].astype(jnp.bfloat16),
                preferred_element_type=jnp.float32)
        + gb1_ref[...], 0.0).astype(jnp.bfloat16)
    h2 = jnp.maximum(
        jnp.dot(h1, g2_ref[...].astype(jnp.bfloat16),
                preferred_element_type=jnp.float32)
        + gb2_ref[...], 0.0).astype(jnp.bfloat16)
    logits = jnp.dot(h2, g3_ref[...].astype(jnp.bfloat16),
                     preferred_element_type=jnp.float32) \
        + gb3_ref[...]
    lane = jax.lax.broadcasted_iota(jnp.int32, logits.shape, 1)
    lm = jnp.where(lane < E, logits, NEG)
    m1 = jnp.max(lm, axis=1, keepdims=True)
    i1 = jnp.min(jnp.where(lm == m1, lane, 10**6), axis=1, keepdims=True)
    lm2 = jnp.where(lane == i1, NEG, lm)
    m2 = jnp.max(lm2, axis=1, keepdims=True)
    i2 = jnp.min(jnp.where(lm2 == m2, lane, 10**6), axis=1, keepdims=True)
    ed = jnp.exp(m2 - m1)
    denom = 1.0 + ed
    p1 = 1.0 / denom
    p2 = ed / denom
    p = jnp.where(lane == i1, p1, jnp.where(lane == i2, p2, 0.0))
    p_ref[...] = p[:, :E]
    tp_ref[:, 0:1] = p1
    tp_ref[:, 1:2] = p2
    i = pl.program_id(0)
    rows = pl.ds(i * BN, BN)
    ti_scr[rows, 0:1] = i1
    ti_scr[rows, 1:2] = i2

    @pl.when(i == N // BN - 1)
    def _():
        e_row = jax.lax.broadcasted_iota(jnp.int32, (1, E), 1)
        ir = jax.lax.broadcasted_iota(jnp.int32, (SB, SB), 0)
        ic = jax.lax.broadcasted_iota(jnp.int32, (SB, SB), 1)
        ltri = (ir > ic).astype(jnp.bfloat16)
        m8r = jax.lax.broadcasted_iota(jnp.int32, (E, E), 0)
        m8c = jax.lax.broadcasted_iota(jnp.int32, (E, E), 1)
        mincl = (m8r <= m8c).astype(jnp.float32)

        def ind(kk, j):
            col = ti_scr[pl.ds(j * SB, SB), kk:kk + 1]
            return (col == e_row).astype(jnp.float32)

        def cnt(kk, j, c):
            return c + jnp.sum(ind(kk, j), axis=0, keepdims=True)

        c = jnp.zeros((1, E), jnp.float32)
        c = lax.fori_loop(0, N // SB, functools.partial(cnt, 0), c)
        c = lax.fori_loop(0, N // SB, functools.partial(cnt, 1), c)
        gp = ((c.astype(jnp.int32) + BM - 1) // BM) * BM
        gpf = gp.astype(jnp.float32)
        cumgp = jnp.dot(gpf, mincl, preferred_element_type=jnp.float32)
        gs = cumgp - gpf

        jbase = jax.lax.broadcasted_iota(jnp.int32, (1, 64), 1) * BM
        cum_i = cumgp.astype(jnp.int32)
        acc = jnp.zeros((1, 64), jnp.int32)
        for e in range(E):
            acc = acc + (jbase >= cum_i[0:1, e:e + 1]).astype(jnp.int32)
        blk_ref[...] = jnp.minimum(acc, E - 1)

        def rank(kk, j, c):
            ib = ind(kk, j)
            r = jnp.dot(ltri, ib.astype(jnp.bfloat16),
                        preferred_element_type=jnp.float32) + c
            slot = jnp.sum(ib * (gs + r), axis=1, keepdims=True)
            pos_ref[pl.ds(j * SB, SB), kk:kk + 1] = slot.astype(jnp.int32)
            return c + jnp.sum(ib, axis=0, keepdims=True)

        c2 = jnp.zeros((1, E), jnp.float32)
        c2 = lax.fori_loop(0, N // SB, functools.partial(rank, 0), c2)
        c2 = lax.fori_loop(0, N // SB, functools.partial(rank, 1), c2)


_CH3 = (N * K) // NW


@functools.lru_cache(maxsize=None)
def _make_sc_scatter():
    @functools.partial(
        pl.kernel,
        mesh=plsc.VectorSubcoreMesh(core_axis_name="c", subcore_axis_name="s"),
        out_type=jax.ShapeDtypeStruct((SPAD, D), jnp.float32),
        scratch_types=[
            pltpu.VMEM((_CH3,), jnp.int32),
            pltpu.VMEM((_CH3, D), jnp.float32),
            pltpu.SemaphoreType.DMA,
        ],
    )
    def body(x_hbm, pos_hbm, xs_hbm, idx_v, rows_v, sem):
        wid = lax.axis_index("s") * NC + lax.axis_index("c")
        base = wid * _CH3
        kq = base // N
        t0 = base - kq * N
        pltpu.sync_copy(pos_hbm.at[kq, pl.ds(t0, _CH3)], idx_v)
        pltpu.sync_copy(x_hbm.at[pl.ds(t0, _CH3), :], rows_v)
        pltpu.async_copy(rows_v, xs_hbm.at[idx_v], sem).wait()

    return body


def _sc_scatter_rows(x, posT):
    return _make_sc_scatter()(x, posT)


def _expert_body(s_ref, xs_ref, w1_ref, b1_ref, w2_ref, b2_ref, out_ref):
    xb = xs_ref[...].astype(jnp.bfloat16)
    he = jnp.dot(xb, w1_ref[0].astype(jnp.bfloat16),
                 preferred_element_type=jnp.float32)
    he = jnp.maximum(he + b1_ref[0], 0.0).astype(jnp.bfloat16)
    out = jnp.dot(he, w2_ref[0].astype(jnp.bfloat16),
                  preferred_element_type=jnp.float32)
    out_ref[...] = out + b2_ref[0]


_CH5 = N // NW


@functools.lru_cache(maxsize=None)
def _make_sc_gather():
    @functools.partial(
        pl.kernel,
        mesh=plsc.VectorSubcoreMesh(core_axis_name="c", subcore_axis_name="s"),
        out_type=(jax.ShapeDtypeStruct((N, O), jnp.float32),
                  jax.ShapeDtypeStruct((N, O), jnp.float32)),
        scratch_types=[
            pltpu.VMEM((_CH5,), jnp.int32),
            pltpu.VMEM((_CH5, O), jnp.float32),
            pltpu.SemaphoreType.DMA,
        ],
    )
    def body(rows_hbm, pos_hbm, r0_hbm, r1_hbm, idx_v, buf_v, sem):
        wid = lax.axis_index("s") * NC + lax.axis_index("c")
        t0 = wid * _CH5
        pltpu.sync_copy(pos_hbm.at[0, pl.ds(t0, _CH5)], idx_v)
        pltpu.async_copy(rows_hbm.at[idx_v], buf_v, sem).wait()
        pltpu.sync_copy(buf_v, r0_hbm.at[pl.ds(t0, _CH5), :])
        pltpu.sync_copy(pos_hbm.at[1, pl.ds(t0, _CH5)], idx_v)
        pltpu.async_copy(rows_hbm.at[idx_v], buf_v, sem).wait()
        pltpu.sync_copy(buf_v, r1_hbm.at[pl.ds(t0, _CH5), :])

    return body


def _sc_gather_out(rows, posT):
    return _make_sc_gather()(rows, posT)


def _combine_body(tp_ref, r0_ref, r1_ref, y_ref):
    y_ref[...] = tp_ref[:, 0:1] * r0_ref[...] + tp_ref[:, 1:2] * r1_ref[...]


@jax.jit
def kernel(x, W1, b1, W2, b2, g1, gb1, g2, gb2, g3, gb3):
    g3p = jnp.zeros((D, 128), jnp.float32).at[:, :E].set(g3)
    gb3p = jnp.zeros((1, 128), jnp.float32).at[0, :E].set(gb3)

    p, topp, pos, blk = pl.pallas_call(
        _gate_body,
        grid=(N // BN,),
        in_specs=[
            pl.BlockSpec((BN, D), lambda i: (i, 0)),
            pl.BlockSpec((D, 4 * D), lambda i: (0, 0)),
            pl.BlockSpec((1, 4 * D), lambda i: (0, 0)),
            pl.BlockSpec((4 * D, D), lambda i: (0, 0)),
            pl.BlockSpec((1, D), lambda i: (0, 0)),
            pl.BlockSpec((D, 128), lambda i: (0, 0)),
            pl.BlockSpec((1, 128), lambda i: (0, 0)),
        ],
        out_specs=[
            pl.BlockSpec((BN, E), lambda i: (i, 0)),
            pl.BlockSpec((BN, K), lambda i: (i, 0)),
            pl.BlockSpec((N, K), lambda i: (0, 0)),
            pl.BlockSpec((1, 64), lambda i: (0, 0)),
        ],
        out_shape=[
            jax.ShapeDtypeStruct((N, E), jnp.float32),
            jax.ShapeDtypeStruct((N, K), jnp.float32),
            jax.ShapeDtypeStruct((N, K), jnp.int32),
            jax.ShapeDtypeStruct((1, 64), jnp.int32),
        ],
        scratch_shapes=[pltpu.VMEM((N, K), jnp.int32)],
    )(x, g1, gb1.reshape(1, 4 * D), g2, gb2.reshape(1, D), g3p, gb3p)

    posT = pos.T
    blk1d = blk.reshape(64)[:NB]

    xs = _sc_scatter_rows(x, posT)

    b1r = b1.reshape(E, 1, H)
    b2r = b2.reshape(E, 1, O)
    out_rows = pl.pallas_call(
        _expert_body,
        grid_spec=pltpu.PrefetchScalarGridSpec(
            num_scalar_prefetch=1,
            grid=(NB,),
            in_specs=[
                pl.BlockSpec((BM, D), lambda j, s: (j, 0)),
                pl.BlockSpec((1, D, H), lambda j, s: (s[j], 0, 0)),
                pl.BlockSpec((1, 1, H), lambda j, s: (s[j], 0, 0)),
                pl.BlockSpec((1, H, O), lambda j, s: (s[j], 0, 0)),
                pl.BlockSpec((1, 1, O), lambda j, s: (s[j], 0, 0)),
            ],
            out_specs=pl.BlockSpec((BM, O), lambda j, s: (j, 0)),
        ),
        out_shape=jax.ShapeDtypeStruct((SPAD, O), jnp.float32),
    )(blk1d, xs, W1, b1r, W2, b2r)

    r0, r1 = _sc_gather_out(out_rows, posT)

    y = pl.pallas_call(
        _combine_body,
        grid=(N // BN,),
        in_specs=[
            pl.BlockSpec((BN, K), lambda i: (i, 0)),
            pl.BlockSpec((BN, O), lambda i: (i, 0)),
            pl.BlockSpec((BN, O), lambda i: (i, 0)),
        ],
        out_specs=pl.BlockSpec((BN, O), lambda i: (i, 0)),
        out_shape=jax.ShapeDtypeStruct((N, O), jnp.float32),
    )(topp, r0, r1)

    return (y, p)
